# baseline (speedup 1.0000x reference)
import jax
import jax.numpy as jnp
from jax.experimental import pallas as pl
from jax.experimental.pallas import tpu as pltpu

_B, _C, _L, _D, _K = 16, 256, 4096, 256, 128
_LC = 2048
_NJ = _L // _LC


def _vq_body(x_ref, we_ref, be_ref, cb_ref, wd_ref, bd_ref,
             out_ref, idx_ref, loss_ref, dcb_ref):
    first = (pl.program_id(0) == 0) & (pl.program_id(1) == 0)

    @pl.when(first)
    def _init():
        dcb = jax.lax.dot_general(
            wd_ref[...], cb_ref[...],
            dimension_numbers=(((1,), (1,)), ((), ()))) + bd_ref[...]
        dcb_ref[...] = dcb.astype(jnp.bfloat16)

    xb = x_ref[0]
    zT = jnp.dot(we_ref[...], xb) + be_ref[...]
    scores = jnp.dot(cb_ref[...], zT)
    cb_sq = jnp.sum(cb_ref[...] * cb_ref[...], axis=1, keepdims=True)
    e = cb_sq - 2.0 * scores

    mine = jnp.min(e, axis=0)
    iota_k = jax.lax.broadcasted_iota(jnp.int32, (_K, _LC), 0)
    idx = jnp.min(jnp.where(e == mine[None, :], iota_k, _K), axis=0)
    idx_ref[0, 0, :] = idx

    onehot = (iota_k == idx[None, :]).astype(jnp.bfloat16)
    out_ref[0] = jax.lax.dot_general(
        dcb_ref[...], onehot, dimension_numbers=(((1,), (0,)), ((), ())),
        preferred_element_type=jnp.float32)

    part = (jnp.sum(zT * zT) + jnp.sum(mine)).reshape(1, 1)

    @pl.when(first)
    def _set():
        loss_ref[...] = part

    @pl.when(jnp.logical_not(first))
    def _acc():
        loss_ref[...] += part


def kernel(x, W_enc, b_enc, codebook, W_dec, b_dec):
    out, idx3, loss_sum = pl.pallas_call(
        _vq_body,
        grid=(_B, _NJ),
        in_specs=[
            pl.BlockSpec((1, _C, _LC), lambda i, j: (i, 0, j)),
            pl.BlockSpec((_D, _C), lambda i, j: (0, 0)),
            pl.BlockSpec((_D, 1), lambda i, j: (0, 0)),
            pl.BlockSpec((_K, _D), lambda i, j: (0, 0)),
            pl.BlockSpec((_C, _D), lambda i, j: (0, 0)),
            pl.BlockSpec((_C, 1), lambda i, j: (0, 0)),
        ],
        out_specs=[
            pl.BlockSpec((1, _C, _LC), lambda i, j: (i, 0, j)),
            pl.BlockSpec((1, 1, _LC), lambda i, j: (i, 0, j)),
            pl.BlockSpec((1, 1), lambda i, j: (0, 0)),
        ],
        out_shape=[
            jax.ShapeDtypeStruct((_B, _C, _L), jnp.float32),
            jax.ShapeDtypeStruct((_B, 1, _L), jnp.int32),
            jax.ShapeDtypeStruct((1, 1), jnp.float32),
        ],
        scratch_shapes=[pltpu.VMEM((_C, _K), jnp.bfloat16)],
    )(x, W_enc, b_enc.reshape(_D, 1), codebook, W_dec, b_dec.reshape(_C, 1))
    indices = idx3.reshape(_B, _L)
    commit_loss = (loss_sum[0, 0] / (_B * _L * _D)).astype(jnp.float32)
    return (out, indices, commit_loss)


# speedup vs baseline: 1.2206x; 1.2206x over previous
import jax
import jax.numpy as jnp
from jax.experimental import pallas as pl
from jax.experimental.pallas import tpu as pltpu

_B, _C, _L, _D, _K = 16, 256, 4096, 256, 128
_NB = 2


def _vq_body(x_ref, we_ref, be_ref, cb_ref, wd_ref, bd_ref,
             out_ref, idx_ref, loss_ref, dcb_ref):
    first = pl.program_id(0) == 0

    @pl.when(first)
    def _init():
        dcb = jax.lax.dot_general(
            wd_ref[...], cb_ref[...],
            dimension_numbers=(((1,), (1,)), ((), ()))) + bd_ref[...]
        dcb_ref[...] = dcb.astype(jnp.bfloat16)

    for _bi in range(_NB):
        _vq_one(x_ref, we_ref, be_ref, cb_ref, out_ref, idx_ref, loss_ref,
                dcb_ref, first & (_bi == 0), _bi)


def _vq_one(x_ref, we_ref, be_ref, cb_ref, out_ref, idx_ref, loss_ref,
            dcb_ref, first, bi):
    xb = x_ref[bi]
    zT = jnp.dot(we_ref[...], xb) + be_ref[...]
    scores = jnp.dot(cb_ref[...], zT)
    cb_sq = jnp.sum(cb_ref[...] * cb_ref[...], axis=1, keepdims=True)
    e = cb_sq - 2.0 * scores

    mine = jnp.min(e, axis=0)
    iota_k = jax.lax.broadcasted_iota(jnp.int32, (_K, _L), 0)
    idx = jnp.min(jnp.where(e == mine[None, :], iota_k, _K), axis=0)
    idx_ref[bi, 0, :] = idx

    onehot = (iota_k == idx[None, :]).astype(jnp.bfloat16)
    out_ref[bi] = jax.lax.dot_general(
        dcb_ref[...], onehot, dimension_numbers=(((1,), (0,)), ((), ())),
        preferred_element_type=jnp.float32)

    part = (jnp.sum(zT * zT) + jnp.sum(mine)).reshape(1, 1)

    @pl.when(first)
    def _set():
        loss_ref[...] = part

    @pl.when(jnp.logical_not(first))
    def _acc():
        loss_ref[...] += part


def kernel(x, W_enc, b_enc, codebook, W_dec, b_dec):
    out, idx3, loss_sum = pl.pallas_call(
        _vq_body,
        grid=(_B // _NB,),
        in_specs=[
            pl.BlockSpec((_NB, _C, _L), lambda i: (i, 0, 0)),
            pl.BlockSpec((_D, _C), lambda i: (0, 0)),
            pl.BlockSpec((_D, 1), lambda i: (0, 0)),
            pl.BlockSpec((_K, _D), lambda i: (0, 0)),
            pl.BlockSpec((_C, _D), lambda i: (0, 0)),
            pl.BlockSpec((_C, 1), lambda i: (0, 0)),
        ],
        out_specs=[
            pl.BlockSpec((_NB, _C, _L), lambda i: (i, 0, 0)),
            pl.BlockSpec((_NB, 1, _L), lambda i: (i, 0, 0)),
            pl.BlockSpec((1, 1), lambda i: (0, 0)),
        ],
        out_shape=[
            jax.ShapeDtypeStruct((_B, _C, _L), jnp.float32),
            jax.ShapeDtypeStruct((_B, 1, _L), jnp.int32),
            jax.ShapeDtypeStruct((1, 1), jnp.float32),
        ],
        scratch_shapes=[pltpu.VMEM((_C, _K), jnp.bfloat16)],
    )(x, W_enc, b_enc.reshape(_D, 1), codebook, W_dec, b_dec.reshape(_C, 1))
    indices = idx3.reshape(_B, _L)
    commit_loss = (loss_sum[0, 0] / (_B * _L * _D)).astype(jnp.float32)
    return (out, indices, commit_loss)
